# baseline (speedup 1.0000x reference)
import functools

import jax
import jax.numpy as jnp
import numpy as np
from jax import lax
from jax.experimental import pallas as pl
from jax.experimental.pallas import tpu as pltpu
from jax.experimental.pallas import tpu_sc as plsc

_B = 1024
_D = 128
_CD = 256
_M = 100000
_TOPK = 16
_TEMP = 0.07
_S = 16
_T = 2048
_NT = (_M + _T - 1) // _T
_MP = _NT * _T
_C = _MP // _S
_NEG = float("-inf")


def _qproj_body(cr_ref, wq_ref, bq_ref, q_ref):
    q = jnp.dot(cr_ref[...], wq_ref[...].T, preferred_element_type=jnp.float32)
    q = q + bq_ref[...]
    n = jnp.sqrt(jnp.sum(q * q, axis=1, keepdims=True))
    q_ref[...] = q / jnp.maximum(n, 1e-12)


def _qproj(cr, wq, bq):
    return pl.pallas_call(
        _qproj_body,
        out_shape=jax.ShapeDtypeStruct((_B, _D), jnp.float32),
    )(cr, wq, bq.reshape(1, _D))


_BB2 = 512
_NB2 = _B // _BB2
_NSTEP = _NB2 * _NT
_G = _C // _S


def _sims_body(q_ref, fq_ref, wk_ref, bk_ref, sid_ref, sq_ref,
               sims_hbm, cmax_ref, sbuf, sem):
    b = pl.program_id(0)
    i = pl.program_id(1)
    step = b * _NT + i
    ph = step % 2
    k = jnp.dot(fq_ref[...], wk_ref[...].T, preferred_element_type=jnp.float32)
    k = k + bk_ref[...]
    n = jnp.sqrt(jnp.sum(k * k, axis=1, keepdims=True))
    k = k / jnp.maximum(n, 1e-12)
    sims = jnp.dot(q_ref[...], k.T, preferred_element_type=jnp.float32)
    col = i * _T + lax.broadcasted_iota(jnp.int32, (1, _T), 1)
    valid = (sq_ref[0] != sid_ref[...]) & (col < _M)
    sims = jnp.where(valid, sims, _NEG)

    def _copy(phase, s):
        return pltpu.make_async_copy(
            sbuf.at[phase],
            sims_hbm.at[pl.ds(s * (_BB2 * _S), _BB2 * _S)],
            sem.at[phase])

    @pl.when(step >= 2)
    def _():
        _copy(ph, step - 2).wait()
    for u in range(_S):
        sbuf[ph, pl.ds(u * _BB2, _BB2)] = sims[:, u * 128:(u + 1) * 128]
    _copy(ph, step).start()
    cmax_ref[...] = jnp.max(sims.reshape(_BB2, _S, _T // _S), axis=1)

    @pl.when(step == _NSTEP - 1)
    def _():
        _copy(1 - ph, step - 1).wait()
        _copy(ph, step).wait()


def _sims_cmax(q, fq_pad, wk, bk, sid, sq_pad):
    return pl.pallas_call(
        _sims_body,
        grid=(_NB2, _NT),
        in_specs=[
            pl.BlockSpec((_BB2, _D), lambda b, i: (b, 0)),
            pl.BlockSpec((_T, _D), lambda b, i: (i, 0)),
            pl.BlockSpec((_D, _D), lambda b, i: (0, 0)),
            pl.BlockSpec((1, _D), lambda b, i: (0, 0)),
            pl.BlockSpec((_BB2, 1), lambda b, i: (b, 0)),
            pl.BlockSpec((1, 1, _T), lambda b, i: (i, 0, 0)),
        ],
        out_specs=[
            pl.BlockSpec(memory_space=pltpu.MemorySpace.HBM),
            pl.BlockSpec((_BB2, _T // _S), lambda b, i: (b, i)),
        ],
        out_shape=[
            jax.ShapeDtypeStruct((_NSTEP * _BB2 * _S, 128), jnp.float32),
            jax.ShapeDtypeStruct((_B, _C), jnp.float32),
        ],
        scratch_shapes=[
            pltpu.VMEM((2, _BB2 * _S, 128), jnp.float32),
            pltpu.SemaphoreType.DMA((2,)),
        ],
    )(q, fq_pad, wk, bk.reshape(1, _D), sid, sq_pad)


_BB3 = 128
_NC = _TOPK * _S


def _iter_topk(x, width, k):
    iota = lax.broadcasted_iota(jnp.int32, (_BB3, width), 1)
    avail = jnp.ones((_BB3, width), jnp.bool_)
    out = []
    for _ in range(k):
        xa = jnp.where(avail, x, _NEG)
        m = jnp.max(xa, axis=1, keepdims=True)
        eq = avail & ((xa == m) | (m == _NEG))
        idx = jnp.min(jnp.where(eq, iota, width), axis=1, keepdims=True)
        out.append(idx)
        avail = avail & (iota != idx)
    return out


def _topchunk_body(cmax_ref, cid_ref):
    sel = _iter_topk(cmax_ref[...], _C, _TOPK)
    cid_ref[...] = jnp.concatenate(sel, axis=1)


def _topchunks(cmax):
    return pl.pallas_call(
        _topchunk_body,
        grid=(_B // _BB3,),
        in_specs=[
            pl.BlockSpec((_BB3, _C), lambda i: (i, 0)),
        ],
        out_specs=pl.BlockSpec((_BB3, _TOPK), lambda i: (i, 0)),
        out_shape=jax.ShapeDtypeStruct((_B, _TOPK), jnp.int32),
    )(cmax)


_NW = 32
_RPW = _B // _NW


def _sc_body(sims_hbm, cids_hbm, fq_hbm, tq_hbm, ie_hbm,
             tv_hbm, nf_hbm, ie_out_hbm,
             cid_v, cbuf, tvbuf, ribuf, tibuf, tgt_v,
             nf_v, iev_v, sem):
    wid = lax.axis_index("s") * 2 + lax.axis_index("c")
    base = wid * _RPW

    def row(r, carry):
        b = base + r
        pltpu.sync_copy(cids_hbm.at[b], cid_v)
        g = cid_v[...]
        tile = lax.shift_right_logical(g, 7)
        lane = g & 127
        step = (b // _BB2) * _NT + tile
        r0 = step * (_BB2 * _S) + (b % _BB2)
        gbase = tile * _T + lane
        iota16 = lax.iota(jnp.int32, _S)

        def start(u, slot):
            return pltpu.async_copy(
                sims_hbm.at[r0 + _BB2 * u], cbuf.at[slot], sem.at[slot])

        h = [start(0, 0), start(1, 1)]
        rv = jnp.full((_S,), _NEG, jnp.float32)
        ri = jnp.zeros((_S,), jnp.int32)
        for u in range(_TOPK):
            h[u % 2].wait()
            cv = plsc.load_gather(cbuf.at[u % 2], [iota16, lane])
            gi = gbase + 128 * u
            if u + 2 < _TOPK:
                h[u % 2] = start(u + 2, u % 2)
            nv, ni = plsc.sort_key_val(cv, gi, descending=True)
            ge = rv >= nv
            mx = jnp.where(ge, rv, nv)
            mi = jnp.where(ge, ri, ni)
            rv, ri = plsc.sort_key_val(mx, mi)
        tvbuf[...] = rv
        ribuf[...] = jnp.clip(ri, 0, _M - 1)
        pltpu.sync_copy(tvbuf, tv_hbm.at[b])
        pltpu.async_copy(fq_hbm.at[ribuf], nf_v, sem.at[0]).wait()
        pltpu.async_copy(tq_hbm.at[ribuf], tgt_v, sem.at[0]).wait()
        tibuf[...] = jnp.maximum(tgt_v[...], 0)
        pltpu.async_copy(ie_hbm.at[tibuf], iev_v, sem.at[0]).wait()
        pltpu.sync_copy(nf_v, nf_hbm.at[b])
        pltpu.sync_copy(iev_v, ie_out_hbm.at[b])
        return carry

    lax.fori_loop(0, _RPW, row, 0)


def _candidates_sc(sims2d, cids, fq, tq, ie):
    fn = functools.partial(
        pl.kernel,
        mesh=plsc.VectorSubcoreMesh(core_axis_name="c", subcore_axis_name="s"),
        compiler_params=pltpu.CompilerParams(needs_layout_passes=False),
        out_type=[
            jax.ShapeDtypeStruct((_B, _TOPK), jnp.float32),
            jax.ShapeDtypeStruct((_B, _TOPK, _D), jnp.float32),
            jax.ShapeDtypeStruct((_B, _TOPK, _D), jnp.float32),
        ],
        scratch_types=[
            pltpu.VMEM((_S,), jnp.int32),
            pltpu.VMEM((2, _S, 128), jnp.float32),
            pltpu.VMEM((_S,), jnp.float32),
            pltpu.VMEM((_S,), jnp.int32),
            pltpu.VMEM((_S,), jnp.int32),
            pltpu.VMEM((_S,), jnp.int32),
            pltpu.VMEM((_S, _D), jnp.float32),
            pltpu.VMEM((_S, _D), jnp.float32),
            pltpu.SemaphoreType.DMA((2,)),
        ],
    )(_sc_body)
    return fn(sims2d, cids, fq, tq, ie)


_BB5 = 128


def _final_body(tv_ref, nf_ref, ie_ref, w1_ref, b1_ref, w2_ref, b2_ref,
                fc_ref, fs_ref, ctx_ref, sum_ref, used_ref):
    tv = tv_ref[...]
    selected = tv > -1e30
    used = jnp.any(selected, axis=1, keepdims=True)
    logits = jnp.where(selected, tv * (1.0 / _TEMP), -1e9)
    m = jnp.max(logits, axis=1, keepdims=True)
    e = jnp.exp(logits - m)
    a = e / jnp.sum(e, axis=1, keepdims=True)
    a = a * selected.astype(jnp.float32)
    a = a / jnp.maximum(jnp.sum(a, axis=1, keepdims=True), 1e-12)

    nf = nf_ref[...]
    ctx = jnp.sum(a[:, :, None] * nf, axis=1)

    su = jnp.concatenate([nf, ie_ref[...]], axis=2).reshape(_BB5 * _TOPK, 2 * _D)
    h = jnp.dot(su, w1_ref[...].T, preferred_element_type=jnp.float32) + b1_ref[...]
    h = 0.5 * h * (1.0 + lax.erf(h * np.float32(1.0 / np.sqrt(2.0))))
    sv = jnp.dot(h, w2_ref[...].T, preferred_element_type=jnp.float32) + b2_ref[...]
    sv = sv.reshape(_BB5, _TOPK, _CD)
    summ = jnp.sum(a[:, :, None] * sv, axis=1)

    ctx_ref[...] = jnp.where(used, ctx, fc_ref[...])
    sum_ref[...] = jnp.where(used, summ, fs_ref[...])
    used_ref[...] = used.astype(jnp.int32)


def _final(tv, nf, iemb, w1, b1, w2, b2, fc, fs):
    return pl.pallas_call(
        _final_body,
        grid=(_B // _BB5,),
        in_specs=[
            pl.BlockSpec((_BB5, _TOPK), lambda i: (i, 0)),
            pl.BlockSpec((_BB5, _TOPK, _D), lambda i: (i, 0, 0)),
            pl.BlockSpec((_BB5, _TOPK, _D), lambda i: (i, 0, 0)),
            pl.BlockSpec((_CD, 2 * _D), lambda i: (0, 0)),
            pl.BlockSpec((1, _CD), lambda i: (0, 0)),
            pl.BlockSpec((_CD, _CD), lambda i: (0, 0)),
            pl.BlockSpec((1, _CD), lambda i: (0, 0)),
            pl.BlockSpec((1, _D), lambda i: (0, 0)),
            pl.BlockSpec((1, _CD), lambda i: (0, 0)),
        ],
        out_specs=[
            pl.BlockSpec((_BB5, _D), lambda i: (i, 0)),
            pl.BlockSpec((_BB5, _CD), lambda i: (i, 0)),
            pl.BlockSpec((_BB5, 1), lambda i: (i, 0)),
        ],
        out_shape=[
            jax.ShapeDtypeStruct((_B, _D), jnp.float32),
            jax.ShapeDtypeStruct((_B, _CD), jnp.float32),
            jax.ShapeDtypeStruct((_B, 1), jnp.int32),
        ],
    )(tv, nf, iemb, w1, b1.reshape(1, _CD), w2, b2.reshape(1, _CD),
      fc.reshape(1, _D), fs.reshape(1, _CD))


def kernel(current_repr, session_ids, item_emb, feature_queue, session_queue,
           target_queue, Wq, bq, Wk, bk, W1, b1, W2, b2,
           fallback_context, fallback_summary):
    q = _qproj(current_repr, Wq, bq)
    fq_pad = jnp.pad(feature_queue, ((0, _MP - _M), (0, 0)))
    sq_pad = jnp.pad(session_queue, (0, _MP - _M)).reshape(_NT, 1, _T)
    sid = session_ids.astype(jnp.int32).reshape(_B, 1)
    sims2d, cmax = _sims_cmax(q, fq_pad, Wk, bk, sid, sq_pad)
    cids = _topchunks(cmax)
    tv, nf, iemb = _candidates_sc(sims2d, cids, feature_queue,
                                  target_queue.astype(jnp.int32), item_emb)
    ctx, summ, used = _final(tv, nf, iemb, W1, b1, W2, b2,
                             fallback_context, fallback_summary)
    return ctx, summ, used[:, 0] != 0


# speedup vs baseline: 1.1611x; 1.1611x over previous
import functools

import jax
import jax.numpy as jnp
import numpy as np
from jax import lax
from jax.experimental import pallas as pl
from jax.experimental.pallas import tpu as pltpu
from jax.experimental.pallas import tpu_sc as plsc

_B = 1024
_D = 128
_CD = 256
_M = 100000
_TOPK = 16
_TEMP = 0.07
_S = 16
_T = 2048
_NT = (_M + _T - 1) // _T
_MP = _NT * _T
_C = _MP // _S
_NEG = float("-inf")


def _qproj_body(cr_ref, wq_ref, bq_ref, q_ref):
    q = jnp.dot(cr_ref[...], wq_ref[...].T, preferred_element_type=jnp.float32)
    q = q + bq_ref[...]
    n = jnp.sqrt(jnp.sum(q * q, axis=1, keepdims=True))
    q_ref[...] = q / jnp.maximum(n, 1e-12)


def _qproj(cr, wq, bq):
    return pl.pallas_call(
        _qproj_body,
        out_shape=jax.ShapeDtypeStruct((_B, _D), jnp.float32),
    )(cr, wq, bq.reshape(1, _D))


_BB2 = 512
_NB2 = _B // _BB2
_NSTEP = _NB2 * _NT
_G = _C // _S


def _sims_body(q_ref, fq_ref, wk_ref, bk_ref, sid_ref, sq_ref,
               sims_hbm, cmax_ref, sbuf, sem):
    b = pl.program_id(0)
    i = pl.program_id(1)
    step = b * _NT + i
    ph = step % 2
    k = jnp.dot(fq_ref[...], wk_ref[...].T, preferred_element_type=jnp.float32)
    k = k + bk_ref[...]
    n = jnp.sqrt(jnp.sum(k * k, axis=1, keepdims=True))
    k = k / jnp.maximum(n, 1e-12)
    sims = jnp.dot(q_ref[...], k.T, preferred_element_type=jnp.float32)
    col = i * _T + lax.broadcasted_iota(jnp.int32, (1, _T), 1)
    valid = (sq_ref[0] != sid_ref[...]) & (col < _M)
    sims = jnp.where(valid, sims, _NEG)

    def _copy(phase, s):
        return pltpu.make_async_copy(
            sbuf.at[phase],
            sims_hbm.at[pl.ds(s * _BB2, _BB2)],
            sem.at[phase])

    @pl.when(step >= 2)
    def _():
        _copy(ph, step - 2).wait()
    sbuf[ph] = sims
    _copy(ph, step).start()
    cmax_ref[...] = jnp.max(sims.reshape(_BB2, _S, _T // _S), axis=1)

    @pl.when(step == _NSTEP - 1)
    def _():
        _copy(1 - ph, step - 1).wait()
        _copy(ph, step).wait()


def _sims_cmax(q, fq_pad, wk, bk, sid, sq_pad):
    return pl.pallas_call(
        _sims_body,
        grid=(_NB2, _NT),
        in_specs=[
            pl.BlockSpec((_BB2, _D), lambda b, i: (b, 0)),
            pl.BlockSpec((_T, _D), lambda b, i: (i, 0)),
            pl.BlockSpec((_D, _D), lambda b, i: (0, 0)),
            pl.BlockSpec((1, _D), lambda b, i: (0, 0)),
            pl.BlockSpec((_BB2, 1), lambda b, i: (b, 0)),
            pl.BlockSpec((1, 1, _T), lambda b, i: (i, 0, 0)),
        ],
        out_specs=[
            pl.BlockSpec(memory_space=pltpu.MemorySpace.HBM),
            pl.BlockSpec((_BB2, _T // _S), lambda b, i: (b, i)),
        ],
        out_shape=[
            jax.ShapeDtypeStruct((_NSTEP * _BB2, _T), jnp.float32),
            jax.ShapeDtypeStruct((_B, _C), jnp.float32),
        ],
        scratch_shapes=[
            pltpu.VMEM((2, _BB2, _T), jnp.float32),
            pltpu.SemaphoreType.DMA((2,)),
        ],
    )(q, fq_pad, wk, bk.reshape(1, _D), sid, sq_pad)


_BB3 = 128
_NC = _TOPK * _S


def _iter_topk(x, width, k):
    iota = lax.broadcasted_iota(jnp.int32, (_BB3, width), 1)
    avail = jnp.ones((_BB3, width), jnp.bool_)
    out = []
    for _ in range(k):
        xa = jnp.where(avail, x, _NEG)
        m = jnp.max(xa, axis=1, keepdims=True)
        eq = avail & ((xa == m) | (m == _NEG))
        idx = jnp.min(jnp.where(eq, iota, width), axis=1, keepdims=True)
        out.append(idx)
        avail = avail & (iota != idx)
    return out


def _topchunk_body(cmax_ref, cid_ref):
    sel = _iter_topk(cmax_ref[...], _C, _TOPK)
    cid_ref[...] = jnp.concatenate(sel, axis=1)


def _topchunks(cmax):
    return pl.pallas_call(
        _topchunk_body,
        grid=(_B // _BB3,),
        in_specs=[
            pl.BlockSpec((_BB3, _C), lambda i: (i, 0)),
        ],
        out_specs=pl.BlockSpec((_BB3, _TOPK), lambda i: (i, 0)),
        out_shape=jax.ShapeDtypeStruct((_B, _TOPK), jnp.int32),
    )(cmax)


_NW = 32
_RPW = _B // _NW


def _sc_body(sims_hbm, cids_hbm, fq_hbm, tq_hbm, ie_hbm,
             tv_hbm, nf_hbm, ie_out_hbm,
             cid_v, cbuf, tvbuf, ribuf, tibuf, tgt_v,
             nf_v, iev_v, sem):
    wid = lax.axis_index("s") * 2 + lax.axis_index("c")
    base = wid * _RPW

    def row(r, carry):
        b = base + r
        pltpu.sync_copy(cids_hbm.at[b], cid_v)
        g = cid_v[...]
        tile = lax.shift_right_logical(g, 7)
        lane = g & 127
        rrow = ((b // _BB2) * _NT + tile) * _BB2 + (b % _BB2)
        gbase = tile * _T + lane
        iota16 = lax.iota(jnp.int32, _S)

        pltpu.async_copy(sims_hbm.at[rrow], cbuf, sem.at[0]).wait()
        rv = jnp.full((_S,), _NEG, jnp.float32)
        ri = jnp.zeros((_S,), jnp.int32)
        for u in range(_TOPK):
            cv = plsc.load_gather(cbuf, [iota16, lane + 128 * u])
            gi = gbase + 128 * u
            nv, ni = plsc.sort_key_val(cv, gi, descending=True)
            ge = rv >= nv
            mx = jnp.where(ge, rv, nv)
            mi = jnp.where(ge, ri, ni)
            rv, ri = plsc.sort_key_val(mx, mi)
        tvbuf[...] = rv
        ribuf[...] = jnp.clip(ri, 0, _M - 1)
        pltpu.sync_copy(tvbuf, tv_hbm.at[b])
        pltpu.async_copy(fq_hbm.at[ribuf], nf_v, sem.at[0]).wait()
        pltpu.async_copy(tq_hbm.at[ribuf], tgt_v, sem.at[0]).wait()
        tibuf[...] = jnp.maximum(tgt_v[...], 0)
        pltpu.async_copy(ie_hbm.at[tibuf], iev_v, sem.at[0]).wait()
        pltpu.sync_copy(nf_v, nf_hbm.at[b])
        pltpu.sync_copy(iev_v, ie_out_hbm.at[b])
        return carry

    lax.fori_loop(0, _RPW, row, 0)


def _candidates_sc(sims2d, cids, fq, tq, ie):
    fn = functools.partial(
        pl.kernel,
        mesh=plsc.VectorSubcoreMesh(core_axis_name="c", subcore_axis_name="s"),
        compiler_params=pltpu.CompilerParams(needs_layout_passes=False),
        out_type=[
            jax.ShapeDtypeStruct((_B, _TOPK), jnp.float32),
            jax.ShapeDtypeStruct((_B, _TOPK, _D), jnp.float32),
            jax.ShapeDtypeStruct((_B, _TOPK, _D), jnp.float32),
        ],
        scratch_types=[
            pltpu.VMEM((_S,), jnp.int32),
            pltpu.VMEM((_S, _T), jnp.float32),
            pltpu.VMEM((_S,), jnp.float32),
            pltpu.VMEM((_S,), jnp.int32),
            pltpu.VMEM((_S,), jnp.int32),
            pltpu.VMEM((_S,), jnp.int32),
            pltpu.VMEM((_S, _D), jnp.float32),
            pltpu.VMEM((_S, _D), jnp.float32),
            pltpu.SemaphoreType.DMA((2,)),
        ],
    )(_sc_body)
    return fn(sims2d, cids, fq, tq, ie)


_BB5 = 128


def _final_body(tv_ref, nf_ref, ie_ref, w1_ref, b1_ref, w2_ref, b2_ref,
                fc_ref, fs_ref, ctx_ref, sum_ref, used_ref):
    tv = tv_ref[...]
    selected = tv > -1e30
    used = jnp.any(selected, axis=1, keepdims=True)
    logits = jnp.where(selected, tv * (1.0 / _TEMP), -1e9)
    m = jnp.max(logits, axis=1, keepdims=True)
    e = jnp.exp(logits - m)
    a = e / jnp.sum(e, axis=1, keepdims=True)
    a = a * selected.astype(jnp.float32)
    a = a / jnp.maximum(jnp.sum(a, axis=1, keepdims=True), 1e-12)

    nf = nf_ref[...]
    ctx = jnp.sum(a[:, :, None] * nf, axis=1)

    su = jnp.concatenate([nf, ie_ref[...]], axis=2).reshape(_BB5 * _TOPK, 2 * _D)
    h = jnp.dot(su, w1_ref[...].T, preferred_element_type=jnp.float32) + b1_ref[...]
    h = 0.5 * h * (1.0 + lax.erf(h * np.float32(1.0 / np.sqrt(2.0))))
    sv = jnp.dot(h, w2_ref[...].T, preferred_element_type=jnp.float32) + b2_ref[...]
    sv = sv.reshape(_BB5, _TOPK, _CD)
    summ = jnp.sum(a[:, :, None] * sv, axis=1)

    ctx_ref[...] = jnp.where(used, ctx, fc_ref[...])
    sum_ref[...] = jnp.where(used, summ, fs_ref[...])
    used_ref[...] = used.astype(jnp.int32)


def _final(tv, nf, iemb, w1, b1, w2, b2, fc, fs):
    return pl.pallas_call(
        _final_body,
        grid=(_B // _BB5,),
        in_specs=[
            pl.BlockSpec((_BB5, _TOPK), lambda i: (i, 0)),
            pl.BlockSpec((_BB5, _TOPK, _D), lambda i: (i, 0, 0)),
            pl.BlockSpec((_BB5, _TOPK, _D), lambda i: (i, 0, 0)),
            pl.BlockSpec((_CD, 2 * _D), lambda i: (0, 0)),
            pl.BlockSpec((1, _CD), lambda i: (0, 0)),
            pl.BlockSpec((_CD, _CD), lambda i: (0, 0)),
            pl.BlockSpec((1, _CD), lambda i: (0, 0)),
            pl.BlockSpec((1, _D), lambda i: (0, 0)),
            pl.BlockSpec((1, _CD), lambda i: (0, 0)),
        ],
        out_specs=[
            pl.BlockSpec((_BB5, _D), lambda i: (i, 0)),
            pl.BlockSpec((_BB5, _CD), lambda i: (i, 0)),
            pl.BlockSpec((_BB5, 1), lambda i: (i, 0)),
        ],
        out_shape=[
            jax.ShapeDtypeStruct((_B, _D), jnp.float32),
            jax.ShapeDtypeStruct((_B, _CD), jnp.float32),
            jax.ShapeDtypeStruct((_B, 1), jnp.int32),
        ],
    )(tv, nf, iemb, w1, b1.reshape(1, _CD), w2, b2.reshape(1, _CD),
      fc.reshape(1, _D), fs.reshape(1, _CD))


def kernel(current_repr, session_ids, item_emb, feature_queue, session_queue,
           target_queue, Wq, bq, Wk, bk, W1, b1, W2, b2,
           fallback_context, fallback_summary):
    q = _qproj(current_repr, Wq, bq)
    fq_pad = jnp.pad(feature_queue, ((0, _MP - _M), (0, 0)))
    sq_pad = jnp.pad(session_queue, (0, _MP - _M)).reshape(_NT, 1, _T)
    sid = session_ids.astype(jnp.int32).reshape(_B, 1)
    sims2d, cmax = _sims_cmax(q, fq_pad, Wk, bk, sid, sq_pad)
    cids = _topchunks(cmax)
    tv, nf, iemb = _candidates_sc(sims2d, cids, feature_queue,
                                  target_queue.astype(jnp.int32), item_emb)
    ctx, summ, used = _final(tv, nf, iemb, W1, b1, W2, b2,
                             fallback_context, fallback_summary)
    return ctx, summ, used[:, 0] != 0


# speedup vs baseline: 1.3824x; 1.1906x over previous
import functools

import jax
import jax.numpy as jnp
import numpy as np
from jax import lax
from jax.experimental import pallas as pl
from jax.experimental.pallas import tpu as pltpu
from jax.experimental.pallas import tpu_sc as plsc

_B = 1024
_D = 128
_CD = 256
_M = 100000
_TOPK = 16
_TEMP = 0.07
_S = 16
_T = 2048
_NT = (_M + _T - 1) // _T
_MP = _NT * _T
_C = _MP // _S
_NEG = float("-inf")
_MASKED = -1e38


def _qproj_body(cr_ref, wq_ref, bq_ref, q_ref):
    q = jnp.dot(cr_ref[...], wq_ref[...].T, preferred_element_type=jnp.float32)
    q = q + bq_ref[...]
    n = jnp.sqrt(jnp.sum(q * q, axis=1, keepdims=True))
    q_ref[...] = q / jnp.maximum(n, 1e-12)


def _qproj(cr, wq, bq):
    return pl.pallas_call(
        _qproj_body,
        out_shape=jax.ShapeDtypeStruct((_B, _D), jnp.float32),
    )(cr, wq, bq.reshape(1, _D))


_BB2 = 512
_NB2 = _B // _BB2
_NSTEP = _NB2 * _NT
_G = _C // _S


def _sims_body(q_ref, fq_ref, wk_ref, bk_ref, sid_ref, sq_ref,
               sims_hbm, cmax_ref, sbuf, sem):
    b = pl.program_id(0)
    i = pl.program_id(1)
    step = b * _NT + i
    ph = step % 2
    k = jnp.dot(fq_ref[...], wk_ref[...].T, preferred_element_type=jnp.float32)
    k = k + bk_ref[...]
    n = jnp.sqrt(jnp.sum(k * k, axis=1, keepdims=True))
    k = k / jnp.maximum(n, 1e-12)
    sims = jnp.dot(q_ref[...], k.T, preferred_element_type=jnp.float32)
    col = i * _T + lax.broadcasted_iota(jnp.int32, (1, _T), 1)
    valid = (sq_ref[0] != sid_ref[...]) & (col < _M)
    sims = jnp.where(valid, sims, _MASKED)

    def _copy(phase, s):
        return pltpu.make_async_copy(
            sbuf.at[phase],
            sims_hbm.at[pl.ds(s * _BB2, _BB2)],
            sem.at[phase])

    @pl.when(step >= 2)
    def _():
        _copy(ph, step - 2).wait()
    sbuf[ph] = sims
    _copy(ph, step).start()
    cmax_ref[...] = jnp.max(sims.reshape(_BB2, _S, _T // _S), axis=1)

    @pl.when(step == _NSTEP - 1)
    def _():
        _copy(1 - ph, step - 1).wait()
        _copy(ph, step).wait()


def _sims_cmax(q, fq_pad, wk, bk, sid, sq_pad):
    return pl.pallas_call(
        _sims_body,
        grid=(_NB2, _NT),
        in_specs=[
            pl.BlockSpec((_BB2, _D), lambda b, i: (b, 0)),
            pl.BlockSpec((_T, _D), lambda b, i: (i, 0)),
            pl.BlockSpec((_D, _D), lambda b, i: (0, 0)),
            pl.BlockSpec((1, _D), lambda b, i: (0, 0)),
            pl.BlockSpec((_BB2, 1), lambda b, i: (b, 0)),
            pl.BlockSpec((1, 1, _T), lambda b, i: (i, 0, 0)),
        ],
        out_specs=[
            pl.BlockSpec(memory_space=pltpu.MemorySpace.HBM),
            pl.BlockSpec((_BB2, _T // _S), lambda b, i: (b, i)),
        ],
        out_shape=[
            jax.ShapeDtypeStruct((_NSTEP * _BB2, _T), jnp.float32),
            jax.ShapeDtypeStruct((_B, _C), jnp.float32),
        ],
        scratch_shapes=[
            pltpu.VMEM((2, _BB2, _T), jnp.float32),
            pltpu.SemaphoreType.DMA((2,)),
        ],
    )(q, fq_pad, wk, bk.reshape(1, _D), sid, sq_pad)


_BB3 = 128
_NC = _TOPK * _S


def _iter_topk(x, width, k):
    iota = lax.broadcasted_iota(jnp.int32, (_BB3, width), 1)
    out = []
    for _ in range(k):
        m = jnp.max(x, axis=1, keepdims=True)
        idx = jnp.min(jnp.where(x == m, iota, width), axis=1, keepdims=True)
        out.append(idx)
        x = jnp.where(iota == idx, _NEG, x)
    return out


def _topchunk_body(cmax_ref, cid_ref):
    sel = _iter_topk(cmax_ref[...], _C, _TOPK)
    cid_ref[...] = jnp.concatenate(sel, axis=1)


def _topchunks(cmax):
    return pl.pallas_call(
        _topchunk_body,
        grid=(_B // _BB3,),
        in_specs=[
            pl.BlockSpec((_BB3, _C), lambda i: (i, 0)),
        ],
        out_specs=pl.BlockSpec((_BB3, _TOPK), lambda i: (i, 0)),
        out_shape=jax.ShapeDtypeStruct((_B, _TOPK), jnp.int32),
    )(cmax)


_NW = 32
_RPW = _B // _NW


def _sc_body(sims_hbm, cids_hbm, fq_hbm, tq_hbm, ie_hbm,
             tv_hbm, nf_hbm, ie_out_hbm,
             cid_v, cbuf, tvbuf, ribuf, tibuf, tgt_v,
             nf_v, iev_v, sem):
    wid = lax.axis_index("s") * 2 + lax.axis_index("c")
    base = wid * _RPW

    def row(r, carry):
        b = base + r
        pltpu.sync_copy(cids_hbm.at[b], cid_v)
        g = cid_v[...]
        tile = lax.shift_right_logical(g, 7)
        lane = g & 127
        rrow = ((b // _BB2) * _NT + tile) * _BB2 + (b % _BB2)
        gbase = tile * _T + lane
        iota16 = lax.iota(jnp.int32, _S)

        pltpu.async_copy(sims_hbm.at[rrow], cbuf, sem.at[0]).wait()
        rv = jnp.full((_S,), _NEG, jnp.float32)
        ri = jnp.zeros((_S,), jnp.int32)
        for u in range(_TOPK):
            cv = plsc.load_gather(cbuf, [iota16, lane + 128 * u])
            gi = gbase + 128 * u
            nv, ni = plsc.sort_key_val(cv, gi, descending=True)
            ge = rv >= nv
            mx = jnp.where(ge, rv, nv)
            mi = jnp.where(ge, ri, ni)
            rv, ri = plsc.sort_key_val(mx, mi)
        tvbuf[...] = rv
        ribuf[...] = jnp.clip(ri, 0, _M - 1)
        pltpu.sync_copy(tvbuf, tv_hbm.at[b])
        pltpu.async_copy(fq_hbm.at[ribuf], nf_v, sem.at[0]).wait()
        pltpu.async_copy(tq_hbm.at[ribuf], tgt_v, sem.at[0]).wait()
        tibuf[...] = jnp.maximum(tgt_v[...], 0)
        pltpu.async_copy(ie_hbm.at[tibuf], iev_v, sem.at[0]).wait()
        pltpu.sync_copy(nf_v, nf_hbm.at[b])
        pltpu.sync_copy(iev_v, ie_out_hbm.at[b])
        return carry

    lax.fori_loop(0, _RPW, row, 0)


def _candidates_sc(sims2d, cids, fq, tq, ie):
    fn = functools.partial(
        pl.kernel,
        mesh=plsc.VectorSubcoreMesh(core_axis_name="c", subcore_axis_name="s"),
        compiler_params=pltpu.CompilerParams(needs_layout_passes=False),
        out_type=[
            jax.ShapeDtypeStruct((_B, _TOPK), jnp.float32),
            jax.ShapeDtypeStruct((_B, _TOPK, _D), jnp.float32),
            jax.ShapeDtypeStruct((_B, _TOPK, _D), jnp.float32),
        ],
        scratch_types=[
            pltpu.VMEM((_S,), jnp.int32),
            pltpu.VMEM((_S, _T), jnp.float32),
            pltpu.VMEM((_S,), jnp.float32),
            pltpu.VMEM((_S,), jnp.int32),
            pltpu.VMEM((_S,), jnp.int32),
            pltpu.VMEM((_S,), jnp.int32),
            pltpu.VMEM((_S, _D), jnp.float32),
            pltpu.VMEM((_S, _D), jnp.float32),
            pltpu.SemaphoreType.DMA((2,)),
        ],
    )(_sc_body)
    return fn(sims2d, cids, fq, tq, ie)


_BB5 = 128


def _final_body(tv_ref, nf_ref, ie_ref, w1_ref, b1_ref, w2_ref, b2_ref,
                fc_ref, fs_ref, ctx_ref, sum_ref, used_ref):
    tv = tv_ref[...]
    selected = tv > -1e30
    used = jnp.any(selected, axis=1, keepdims=True)
    logits = jnp.where(selected, tv * (1.0 / _TEMP), -1e9)
    m = jnp.max(logits, axis=1, keepdims=True)
    e = jnp.exp(logits - m)
    a = e / jnp.sum(e, axis=1, keepdims=True)
    a = a * selected.astype(jnp.float32)
    a = a / jnp.maximum(jnp.sum(a, axis=1, keepdims=True), 1e-12)

    nf = nf_ref[...]
    ctx = jnp.sum(a[:, :, None] * nf, axis=1)

    su = jnp.concatenate([nf, ie_ref[...]], axis=2).reshape(_BB5 * _TOPK, 2 * _D)
    h = jnp.dot(su, w1_ref[...].T, preferred_element_type=jnp.float32) + b1_ref[...]
    h = 0.5 * h * (1.0 + lax.erf(h * np.float32(1.0 / np.sqrt(2.0))))
    sv = jnp.dot(h, w2_ref[...].T, preferred_element_type=jnp.float32) + b2_ref[...]
    sv = sv.reshape(_BB5, _TOPK, _CD)
    summ = jnp.sum(a[:, :, None] * sv, axis=1)

    ctx_ref[...] = jnp.where(used, ctx, fc_ref[...])
    sum_ref[...] = jnp.where(used, summ, fs_ref[...])
    used_ref[...] = used.astype(jnp.int32)


def _final(tv, nf, iemb, w1, b1, w2, b2, fc, fs):
    return pl.pallas_call(
        _final_body,
        grid=(_B // _BB5,),
        in_specs=[
            pl.BlockSpec((_BB5, _TOPK), lambda i: (i, 0)),
            pl.BlockSpec((_BB5, _TOPK, _D), lambda i: (i, 0, 0)),
            pl.BlockSpec((_BB5, _TOPK, _D), lambda i: (i, 0, 0)),
            pl.BlockSpec((_CD, 2 * _D), lambda i: (0, 0)),
            pl.BlockSpec((1, _CD), lambda i: (0, 0)),
            pl.BlockSpec((_CD, _CD), lambda i: (0, 0)),
            pl.BlockSpec((1, _CD), lambda i: (0, 0)),
            pl.BlockSpec((1, _D), lambda i: (0, 0)),
            pl.BlockSpec((1, _CD), lambda i: (0, 0)),
        ],
        out_specs=[
            pl.BlockSpec((_BB5, _D), lambda i: (i, 0)),
            pl.BlockSpec((_BB5, _CD), lambda i: (i, 0)),
            pl.BlockSpec((_BB5, 1), lambda i: (i, 0)),
        ],
        out_shape=[
            jax.ShapeDtypeStruct((_B, _D), jnp.float32),
            jax.ShapeDtypeStruct((_B, _CD), jnp.float32),
            jax.ShapeDtypeStruct((_B, 1), jnp.int32),
        ],
    )(tv, nf, iemb, w1, b1.reshape(1, _CD), w2, b2.reshape(1, _CD),
      fc.reshape(1, _D), fs.reshape(1, _CD))


def kernel(current_repr, session_ids, item_emb, feature_queue, session_queue,
           target_queue, Wq, bq, Wk, bk, W1, b1, W2, b2,
           fallback_context, fallback_summary):
    q = _qproj(current_repr, Wq, bq)
    fq_pad = jnp.pad(feature_queue, ((0, _MP - _M), (0, 0)))
    sq_pad = jnp.pad(session_queue, (0, _MP - _M)).reshape(_NT, 1, _T)
    sid = session_ids.astype(jnp.int32).reshape(_B, 1)
    sims2d, cmax = _sims_cmax(q, fq_pad, Wk, bk, sid, sq_pad)
    cids = _topchunks(cmax)
    tv, nf, iemb = _candidates_sc(sims2d, cids, feature_queue,
                                  target_queue.astype(jnp.int32), item_emb)
    ctx, summ, used = _final(tv, nf, iemb, W1, b1, W2, b2,
                             fallback_context, fallback_summary)
    return ctx, summ, used[:, 0] != 0


# speedup vs baseline: 1.4525x; 1.0507x over previous
import functools

import jax
import jax.numpy as jnp
import numpy as np
from jax import lax
from jax.experimental import pallas as pl
from jax.experimental.pallas import tpu as pltpu
from jax.experimental.pallas import tpu_sc as plsc

_B = 1024
_D = 128
_CD = 256
_M = 100000
_TOPK = 16
_TEMP = 0.07
_S = 16
_T = 2048
_NT = (_M + _T - 1) // _T
_MP = _NT * _T
_C = _MP // _S
_NEG = float("-inf")
_MASKED = -1e38


def _qproj_body(cr_ref, wq_ref, bq_ref, q_ref):
    q = jnp.dot(cr_ref[...], wq_ref[...].T, preferred_element_type=jnp.float32)
    q = q + bq_ref[...]
    n = jnp.sqrt(jnp.sum(q * q, axis=1, keepdims=True))
    q_ref[...] = q / jnp.maximum(n, 1e-12)


def _qproj(cr, wq, bq):
    return pl.pallas_call(
        _qproj_body,
        out_shape=jax.ShapeDtypeStruct((_B, _D), jnp.float32),
    )(cr, wq, bq.reshape(1, _D))


_BB2 = 512
_NB2 = _B // _BB2
_NSTEP = _NB2 * _NT
_G = _C // _S


def _sims_body(q_ref, fq_ref, wk_ref, bk_ref, sid_ref, sq_ref,
               sims_hbm, cmax_ref, sbuf, sem):
    b = pl.program_id(0)
    i = pl.program_id(1)
    step = b * _NT + i
    ph = step % 2
    k = jnp.dot(fq_ref[...], wk_ref[...].T, preferred_element_type=jnp.float32)
    k = k + bk_ref[...]
    n = jnp.sqrt(jnp.sum(k * k, axis=1, keepdims=True))
    k = k / jnp.maximum(n, 1e-12)
    sims = jnp.dot(q_ref[...], k.T, preferred_element_type=jnp.float32)
    col = i * _T + lax.broadcasted_iota(jnp.int32, (1, _T), 1)
    valid = (sq_ref[0] != sid_ref[...]) & (col < _M)
    sims = jnp.where(valid, sims, _MASKED)

    def _copy(phase, s):
        return pltpu.make_async_copy(
            sbuf.at[phase],
            sims_hbm.at[pl.ds(s * _BB2, _BB2)],
            sem.at[phase])

    @pl.when(step >= 2)
    def _():
        _copy(ph, step - 2).wait()
    sbuf[ph] = sims
    _copy(ph, step).start()
    cmax_ref[...] = jnp.max(sims.reshape(_BB2, _S, _T // _S), axis=1)

    @pl.when(step == _NSTEP - 1)
    def _():
        _copy(1 - ph, step - 1).wait()
        _copy(ph, step).wait()


def _sims_cmax(q, fq_pad, wk, bk, sid, sq_pad):
    return pl.pallas_call(
        _sims_body,
        grid=(_NB2, _NT),
        in_specs=[
            pl.BlockSpec((_BB2, _D), lambda b, i: (b, 0)),
            pl.BlockSpec((_T, _D), lambda b, i: (i, 0)),
            pl.BlockSpec((_D, _D), lambda b, i: (0, 0)),
            pl.BlockSpec((1, _D), lambda b, i: (0, 0)),
            pl.BlockSpec((_BB2, 1), lambda b, i: (b, 0)),
            pl.BlockSpec((1, 1, _T), lambda b, i: (i, 0, 0)),
        ],
        out_specs=[
            pl.BlockSpec(memory_space=pltpu.MemorySpace.HBM),
            pl.BlockSpec((_BB2, _T // _S), lambda b, i: (b, i)),
        ],
        out_shape=[
            jax.ShapeDtypeStruct((_NSTEP * _BB2, _T), jnp.float32),
            jax.ShapeDtypeStruct((_B, _C), jnp.float32),
        ],
        scratch_shapes=[
            pltpu.VMEM((2, _BB2, _T), jnp.float32),
            pltpu.SemaphoreType.DMA((2,)),
        ],
    )(q, fq_pad, wk, bk.reshape(1, _D), sid, sq_pad)


_BB3 = 128
_NC = _TOPK * _S


def _iter_topk(x, width, k):
    iota = lax.broadcasted_iota(jnp.int32, (_BB3, width), 1)
    out = []
    for _ in range(k):
        m = jnp.max(x, axis=1, keepdims=True)
        idx = jnp.min(jnp.where(x == m, iota, width), axis=1, keepdims=True)
        out.append(idx)
        x = jnp.where(iota == idx, _NEG, x)
    return out


def _topchunk_body(cmax_ref, cid_ref):
    sel = _iter_topk(cmax_ref[...], _C, _TOPK)
    cid_ref[...] = jnp.concatenate(sel, axis=1)


def _topchunks(cmax):
    nrows = cmax.shape[0]
    return pl.pallas_call(
        _topchunk_body,
        grid=(nrows // _BB3,),
        in_specs=[
            pl.BlockSpec((_BB3, _C), lambda i: (i, 0)),
        ],
        out_specs=pl.BlockSpec((_BB3, _TOPK), lambda i: (i, 0)),
        out_shape=jax.ShapeDtypeStruct((nrows, _TOPK), jnp.int32),
    )(cmax)


_NW = 32


def _sc_body_for(off, rpw):
    def _sc_body(sims_hbm, cids_hbm, fq_hbm, tq_hbm, ie_hbm,
                 tv_hbm, nf_hbm, ie_out_hbm,
                 cid_v, cbuf, tvbuf, ribuf, tibuf, tgt_v,
                 nf_v, iev_v, sem):
        wid = lax.axis_index("s") * 2 + lax.axis_index("c")
        base = wid * rpw

        def row(r, carry):
            b = base + r
            pltpu.sync_copy(cids_hbm.at[b], cid_v)
            g = cid_v[...]
            tile = lax.shift_right_logical(g, 7)
            lane = g & 127
            rrow = (((b + off) // _BB2) * _NT + tile) * _BB2 + (b + off) % _BB2
            gbase = tile * _T + lane
            iota16 = lax.iota(jnp.int32, _S)

            pltpu.async_copy(sims_hbm.at[rrow], cbuf, sem.at[0]).wait()
            rv = jnp.full((_S,), _NEG, jnp.float32)
            ri = jnp.zeros((_S,), jnp.int32)
            for u in range(_TOPK):
                cv = plsc.load_gather(cbuf, [iota16, lane + 128 * u])
                gi = gbase + 128 * u
                nv, ni = plsc.sort_key_val(cv, gi, descending=True)
                ge = rv >= nv
                mx = jnp.where(ge, rv, nv)
                mi = jnp.where(ge, ri, ni)
                rv, ri = plsc.sort_key_val(mx, mi)
            tvbuf[...] = rv
            ribuf[...] = jnp.clip(ri, 0, _M - 1)
            pltpu.sync_copy(tvbuf, tv_hbm.at[b])
            pltpu.async_copy(fq_hbm.at[ribuf], nf_v, sem.at[0]).wait()
            pltpu.async_copy(tq_hbm.at[ribuf], tgt_v, sem.at[0]).wait()
            tibuf[...] = jnp.maximum(tgt_v[...], 0)
            pltpu.async_copy(ie_hbm.at[tibuf], iev_v, sem.at[0]).wait()
            pltpu.sync_copy(nf_v, nf_hbm.at[b])
            pltpu.sync_copy(iev_v, ie_out_hbm.at[b])
            return carry

        lax.fori_loop(0, rpw, row, 0)
    return _sc_body


def _candidates_sc(sims2d, cids, fq, tq, ie, off):
    nrows = cids.shape[0]
    fn = functools.partial(
        pl.kernel,
        mesh=plsc.VectorSubcoreMesh(core_axis_name="c", subcore_axis_name="s"),
        compiler_params=pltpu.CompilerParams(needs_layout_passes=False),
        out_type=[
            jax.ShapeDtypeStruct((nrows, _TOPK), jnp.float32),
            jax.ShapeDtypeStruct((nrows, _TOPK, _D), jnp.float32),
            jax.ShapeDtypeStruct((nrows, _TOPK, _D), jnp.float32),
        ],
        scratch_types=[
            pltpu.VMEM((_S,), jnp.int32),
            pltpu.VMEM((_S, _T), jnp.float32),
            pltpu.VMEM((_S,), jnp.float32),
            pltpu.VMEM((_S,), jnp.int32),
            pltpu.VMEM((_S,), jnp.int32),
            pltpu.VMEM((_S,), jnp.int32),
            pltpu.VMEM((_S, _D), jnp.float32),
            pltpu.VMEM((_S, _D), jnp.float32),
            pltpu.SemaphoreType.DMA((2,)),
        ],
    )(_sc_body_for(off, nrows // _NW))
    return fn(sims2d, cids, fq, tq, ie)


_BB5 = 128


def _final_body(tv_ref, nf_ref, ie_ref, w1_ref, b1_ref, w2_ref, b2_ref,
                fc_ref, fs_ref, ctx_ref, sum_ref, used_ref):
    tv = tv_ref[...]
    selected = tv > -1e30
    used = jnp.any(selected, axis=1, keepdims=True)
    logits = jnp.where(selected, tv * (1.0 / _TEMP), -1e9)
    m = jnp.max(logits, axis=1, keepdims=True)
    e = jnp.exp(logits - m)
    a = e / jnp.sum(e, axis=1, keepdims=True)
    a = a * selected.astype(jnp.float32)
    a = a / jnp.maximum(jnp.sum(a, axis=1, keepdims=True), 1e-12)

    nf = nf_ref[...]
    ctx = jnp.sum(a[:, :, None] * nf, axis=1)

    su = jnp.concatenate([nf, ie_ref[...]], axis=2).reshape(_BB5 * _TOPK, 2 * _D)
    h = jnp.dot(su, w1_ref[...].T, preferred_element_type=jnp.float32) + b1_ref[...]
    h = 0.5 * h * (1.0 + lax.erf(h * np.float32(1.0 / np.sqrt(2.0))))
    sv = jnp.dot(h, w2_ref[...].T, preferred_element_type=jnp.float32) + b2_ref[...]
    sv = sv.reshape(_BB5, _TOPK, _CD)
    summ = jnp.sum(a[:, :, None] * sv, axis=1)

    ctx_ref[...] = jnp.where(used, ctx, fc_ref[...])
    sum_ref[...] = jnp.where(used, summ, fs_ref[...])
    used_ref[...] = used.astype(jnp.int32)


def _final(tv, nf, iemb, w1, b1, w2, b2, fc, fs):
    nrows = tv.shape[0]
    return pl.pallas_call(
        _final_body,
        grid=(nrows // _BB5,),
        in_specs=[
            pl.BlockSpec((_BB5, _TOPK), lambda i: (i, 0)),
            pl.BlockSpec((_BB5, _TOPK, _D), lambda i: (i, 0, 0)),
            pl.BlockSpec((_BB5, _TOPK, _D), lambda i: (i, 0, 0)),
            pl.BlockSpec((_CD, 2 * _D), lambda i: (0, 0)),
            pl.BlockSpec((1, _CD), lambda i: (0, 0)),
            pl.BlockSpec((_CD, _CD), lambda i: (0, 0)),
            pl.BlockSpec((1, _CD), lambda i: (0, 0)),
            pl.BlockSpec((1, _D), lambda i: (0, 0)),
            pl.BlockSpec((1, _CD), lambda i: (0, 0)),
        ],
        out_specs=[
            pl.BlockSpec((_BB5, _D), lambda i: (i, 0)),
            pl.BlockSpec((_BB5, _CD), lambda i: (i, 0)),
            pl.BlockSpec((_BB5, 1), lambda i: (i, 0)),
        ],
        out_shape=[
            jax.ShapeDtypeStruct((nrows, _D), jnp.float32),
            jax.ShapeDtypeStruct((nrows, _CD), jnp.float32),
            jax.ShapeDtypeStruct((nrows, 1), jnp.int32),
        ],
    )(tv, nf, iemb, w1, b1.reshape(1, _CD), w2, b2.reshape(1, _CD),
      fc.reshape(1, _D), fs.reshape(1, _CD))


def kernel(current_repr, session_ids, item_emb, feature_queue, session_queue,
           target_queue, Wq, bq, Wk, bk, W1, b1, W2, b2,
           fallback_context, fallback_summary):
    q = _qproj(current_repr, Wq, bq)
    fq_pad = jnp.pad(feature_queue, ((0, _MP - _M), (0, 0)))
    sq_pad = jnp.pad(session_queue, (0, _MP - _M)).reshape(_NT, 1, _T)
    sid = session_ids.astype(jnp.int32).reshape(_B, 1)
    sims2d, cmax = _sims_cmax(q, fq_pad, Wk, bk, sid, sq_pad)
    tq32 = target_queue.astype(jnp.int32)
    outs = []
    for h in range(2):
        rows = slice(h * _BB2, (h + 1) * _BB2)
        cids = _topchunks(cmax[rows])
        tv, nf, iemb = _candidates_sc(sims2d, cids, feature_queue,
                                      tq32, item_emb, h * _BB2)
        outs.append(_final(tv, nf, iemb, W1, b1, W2, b2,
                           fallback_context, fallback_summary))
    ctx = jnp.concatenate([o[0] for o in outs])
    summ = jnp.concatenate([o[1] for o in outs])
    used = jnp.concatenate([o[2] for o in outs])
    return ctx, summ, used[:, 0] != 0


# speedup vs baseline: 1.5002x; 1.0329x over previous
import functools

import jax
import jax.numpy as jnp
import numpy as np
from jax import lax
from jax.experimental import pallas as pl
from jax.experimental.pallas import tpu as pltpu
from jax.experimental.pallas import tpu_sc as plsc

_B = 1024
_D = 128
_CD = 256
_M = 100000
_TOPK = 16
_TEMP = 0.07
_S = 16
_T = 2048
_NT = (_M + _T - 1) // _T
_MP = _NT * _T
_C = _MP // _S
_NEG = float("-inf")
_MASKED = -1e38


def _qproj_body(cr_ref, wq_ref, bq_ref, q_ref):
    q = jnp.dot(cr_ref[...], wq_ref[...].T, preferred_element_type=jnp.float32)
    q = q + bq_ref[...]
    n = jnp.sqrt(jnp.sum(q * q, axis=1, keepdims=True))
    q_ref[...] = q / jnp.maximum(n, 1e-12)


def _qproj(cr, wq, bq):
    return pl.pallas_call(
        _qproj_body,
        out_shape=jax.ShapeDtypeStruct((_B, _D), jnp.float32),
    )(cr, wq, bq.reshape(1, _D))


_BB2 = 512
_NB2 = _B // _BB2
_NSTEP = _NB2 * _NT
_G = _C // _S


def _sims_body(q_ref, fq_ref, wk_ref, bk_ref, sid_ref, sq_ref,
               sims_hbm, cmax_ref, sbuf, sem):
    b = pl.program_id(0)
    i = pl.program_id(1)
    step = b * _NT + i
    ph = step % 2
    k = jnp.dot(fq_ref[...], wk_ref[...].T, preferred_element_type=jnp.float32)
    k = k + bk_ref[...]
    n = jnp.sqrt(jnp.sum(k * k, axis=1, keepdims=True))
    k = k / jnp.maximum(n, 1e-12)
    sims = jnp.dot(q_ref[...], k.T, preferred_element_type=jnp.float32)
    col = i * _T + lax.broadcasted_iota(jnp.int32, (1, _T), 1)
    valid = (sq_ref[0] != sid_ref[...]) & (col < _M)
    sims = jnp.where(valid, sims, _MASKED)

    def _copy(phase, s):
        return pltpu.make_async_copy(
            sbuf.at[phase],
            sims_hbm.at[pl.ds(s * _BB2, _BB2)],
            sem.at[phase])

    @pl.when(step >= 2)
    def _():
        _copy(ph, step - 2).wait()
    sbuf[ph] = sims
    _copy(ph, step).start()
    cmax_ref[...] = jnp.max(sims.reshape(_BB2, _S, _T // _S), axis=1)

    @pl.when(step == _NSTEP - 1)
    def _():
        _copy(1 - ph, step - 1).wait()
        _copy(ph, step).wait()


def _sims_cmax(q, fq_pad, wk, bk, sid, sq_pad):
    return pl.pallas_call(
        _sims_body,
        grid=(_NB2, _NT),
        in_specs=[
            pl.BlockSpec((_BB2, _D), lambda b, i: (b, 0)),
            pl.BlockSpec((_T, _D), lambda b, i: (i, 0)),
            pl.BlockSpec((_D, _D), lambda b, i: (0, 0)),
            pl.BlockSpec((1, _D), lambda b, i: (0, 0)),
            pl.BlockSpec((_BB2, 1), lambda b, i: (b, 0)),
            pl.BlockSpec((1, 1, _T), lambda b, i: (i, 0, 0)),
        ],
        out_specs=[
            pl.BlockSpec(memory_space=pltpu.MemorySpace.HBM),
            pl.BlockSpec((_BB2, _T // _S), lambda b, i: (b, i)),
        ],
        out_shape=[
            jax.ShapeDtypeStruct((_NSTEP * _BB2, _T), jnp.float32),
            jax.ShapeDtypeStruct((_B, _C), jnp.float32),
        ],
        scratch_shapes=[
            pltpu.VMEM((2, _BB2, _T), jnp.float32),
            pltpu.SemaphoreType.DMA((2,)),
        ],
    )(q, fq_pad, wk, bk.reshape(1, _D), sid, sq_pad)


_BB3 = 128
_NC = _TOPK * _S


def _iter_topk(x, width, k):
    iota = lax.broadcasted_iota(jnp.int32, (_BB3, width), 1)
    out = []
    for _ in range(k):
        m = jnp.max(x, axis=1, keepdims=True)
        idx = jnp.min(jnp.where(x == m, iota, width), axis=1, keepdims=True)
        out.append(idx)
        x = jnp.where(iota == idx, _NEG, x)
    return out


def _topchunk_body(cmax_ref, cid_ref):
    sel = _iter_topk(cmax_ref[...], _C, _TOPK)
    cid_ref[...] = jnp.concatenate(sel, axis=1)


def _topchunks(cmax):
    nrows = cmax.shape[0]
    return pl.pallas_call(
        _topchunk_body,
        grid=(nrows // _BB3,),
        in_specs=[
            pl.BlockSpec((_BB3, _C), lambda i: (i, 0)),
        ],
        out_specs=pl.BlockSpec((_BB3, _TOPK), lambda i: (i, 0)),
        out_shape=jax.ShapeDtypeStruct((nrows, _TOPK), jnp.int32),
    )(cmax)


_NW = 32


def _sc_body_for(off, rpw):
    def _sc_body(sims_hbm, cids_hbm, fq_hbm, tq_hbm, ie_hbm,
                 tv_hbm, nf_hbm, ie_out_hbm,
                 cid_v, cbuf, tvbuf, ribuf, tibuf, tgt_v,
                 nf_v, iev_v, sem):
        wid = lax.axis_index("s") * 2 + lax.axis_index("c")
        base = wid * rpw

        def row(r, carry):
            b = base + r
            pltpu.sync_copy(cids_hbm.at[b], cid_v)
            g = cid_v[...]
            tile = lax.shift_right_logical(g, 7)
            lane = g & 127
            rrow = (((b + off) // _BB2) * _NT + tile) * _BB2 + (b + off) % _BB2
            gbase = tile * _T + lane
            iota16 = lax.iota(jnp.int32, _S)

            pltpu.async_copy(sims_hbm.at[rrow], cbuf, sem.at[0]).wait()
            rv = jnp.full((_S,), _NEG, jnp.float32)
            ri = jnp.zeros((_S,), jnp.int32)
            for u in range(_TOPK):
                cv = plsc.load_gather(cbuf, [iota16, lane + 128 * u])
                gi = gbase + 128 * u
                nv, ni = plsc.sort_key_val(cv, gi, descending=True)
                ge = rv >= nv
                mx = jnp.where(ge, rv, nv)
                mi = jnp.where(ge, ri, ni)
                rv, ri = plsc.sort_key_val(mx, mi)
            tvbuf[...] = rv
            ribuf[...] = jnp.clip(ri, 0, _M - 1)
            pltpu.sync_copy(tvbuf, tv_hbm.at[b])
            pltpu.async_copy(fq_hbm.at[ribuf], nf_v, sem.at[0]).wait()
            pltpu.async_copy(tq_hbm.at[ribuf], tgt_v, sem.at[0]).wait()
            tibuf[...] = jnp.maximum(tgt_v[...], 0)
            pltpu.async_copy(ie_hbm.at[tibuf], iev_v, sem.at[0]).wait()
            pltpu.sync_copy(nf_v, nf_hbm.at[b])
            pltpu.sync_copy(iev_v, ie_out_hbm.at[b])
            return carry

        lax.fori_loop(0, rpw, row, 0)
    return _sc_body


def _candidates_sc(sims2d, cids, fq, tq, ie, off):
    nrows = cids.shape[0]
    fn = functools.partial(
        pl.kernel,
        mesh=plsc.VectorSubcoreMesh(core_axis_name="c", subcore_axis_name="s"),
        compiler_params=pltpu.CompilerParams(needs_layout_passes=False),
        out_type=[
            jax.ShapeDtypeStruct((nrows, _TOPK), jnp.float32),
            jax.ShapeDtypeStruct((nrows, _TOPK, _D), jnp.float32),
            jax.ShapeDtypeStruct((nrows, _TOPK, _D), jnp.float32),
        ],
        scratch_types=[
            pltpu.VMEM((_S,), jnp.int32),
            pltpu.VMEM((_S, _T), jnp.float32),
            pltpu.VMEM((_S,), jnp.float32),
            pltpu.VMEM((_S,), jnp.int32),
            pltpu.VMEM((_S,), jnp.int32),
            pltpu.VMEM((_S,), jnp.int32),
            pltpu.VMEM((_S, _D), jnp.float32),
            pltpu.VMEM((_S, _D), jnp.float32),
            pltpu.SemaphoreType.DMA((2,)),
        ],
    )(_sc_body_for(off, nrows // _NW))
    return fn(sims2d, cids, fq, tq, ie)


_BB5 = 128


def _final_body(tv_ref, nf_ref, ie_ref, w1_ref, b1_ref, w2_ref, b2_ref,
                fc_ref, fs_ref, ctx_ref, sum_ref, used_ref):
    tv = tv_ref[...]
    selected = tv > -1e30
    used = jnp.any(selected, axis=1, keepdims=True)
    logits = jnp.where(selected, tv * (1.0 / _TEMP), -1e9)
    m = jnp.max(logits, axis=1, keepdims=True)
    e = jnp.exp(logits - m)
    a = e / jnp.sum(e, axis=1, keepdims=True)
    a = a * selected.astype(jnp.float32)
    a = a / jnp.maximum(jnp.sum(a, axis=1, keepdims=True), 1e-12)

    nf = nf_ref[...]
    ctx = jnp.sum(a[:, :, None] * nf, axis=1)

    su = jnp.concatenate([nf, ie_ref[...]], axis=2).reshape(_BB5 * _TOPK, 2 * _D)
    h = jnp.dot(su, w1_ref[...].T, preferred_element_type=jnp.float32) + b1_ref[...]
    h = 0.5 * h * (1.0 + lax.erf(h * np.float32(1.0 / np.sqrt(2.0))))
    sv = jnp.dot(h, w2_ref[...].T, preferred_element_type=jnp.float32) + b2_ref[...]
    sv = sv.reshape(_BB5, _TOPK, _CD)
    summ = jnp.sum(a[:, :, None] * sv, axis=1)

    ctx_ref[...] = jnp.where(used, ctx, fc_ref[...])
    sum_ref[...] = jnp.where(used, summ, fs_ref[...])
    used_ref[...] = used.astype(jnp.int32)


def _final(tv, nf, iemb, w1, b1, w2, b2, fc, fs):
    nrows = tv.shape[0]
    return pl.pallas_call(
        _final_body,
        grid=(nrows // _BB5,),
        in_specs=[
            pl.BlockSpec((_BB5, _TOPK), lambda i: (i, 0)),
            pl.BlockSpec((_BB5, _TOPK, _D), lambda i: (i, 0, 0)),
            pl.BlockSpec((_BB5, _TOPK, _D), lambda i: (i, 0, 0)),
            pl.BlockSpec((_CD, 2 * _D), lambda i: (0, 0)),
            pl.BlockSpec((1, _CD), lambda i: (0, 0)),
            pl.BlockSpec((_CD, _CD), lambda i: (0, 0)),
            pl.BlockSpec((1, _CD), lambda i: (0, 0)),
            pl.BlockSpec((1, _D), lambda i: (0, 0)),
            pl.BlockSpec((1, _CD), lambda i: (0, 0)),
        ],
        out_specs=[
            pl.BlockSpec((_BB5, _D), lambda i: (i, 0)),
            pl.BlockSpec((_BB5, _CD), lambda i: (i, 0)),
            pl.BlockSpec((_BB5, 1), lambda i: (i, 0)),
        ],
        out_shape=[
            jax.ShapeDtypeStruct((nrows, _D), jnp.float32),
            jax.ShapeDtypeStruct((nrows, _CD), jnp.float32),
            jax.ShapeDtypeStruct((nrows, 1), jnp.int32),
        ],
    )(tv, nf, iemb, w1, b1.reshape(1, _CD), w2, b2.reshape(1, _CD),
      fc.reshape(1, _D), fs.reshape(1, _CD))


def kernel(current_repr, session_ids, item_emb, feature_queue, session_queue,
           target_queue, Wq, bq, Wk, bk, W1, b1, W2, b2,
           fallback_context, fallback_summary):
    q = _qproj(current_repr, Wq, bq)
    fq_pad = jnp.pad(feature_queue, ((0, _MP - _M), (0, 0)))
    sq_pad = jnp.pad(session_queue, (0, _MP - _M)).reshape(_NT, 1, _T)
    sid = session_ids.astype(jnp.int32).reshape(_B, 1)
    sims2d, cmax = _sims_cmax(q, fq_pad, Wk, bk, sid, sq_pad)
    tq32 = target_queue.astype(jnp.int32)
    outs = []
    nsplit = 4
    rb = _B // nsplit
    for h in range(nsplit):
        rows = slice(h * rb, (h + 1) * rb)
        cids = _topchunks(cmax[rows])
        tv, nf, iemb = _candidates_sc(sims2d, cids, feature_queue,
                                      tq32, item_emb, h * rb)
        outs.append(_final(tv, nf, iemb, W1, b1, W2, b2,
                           fallback_context, fallback_summary))
    ctx = jnp.concatenate([o[0] for o in outs])
    summ = jnp.concatenate([o[1] for o in outs])
    used = jnp.concatenate([o[2] for o in outs])
    return ctx, summ, used[:, 0] != 0
